# baseline (speedup 1.0000x reference)
import functools

import jax
import jax.numpy as jnp
import numpy as np
from jax import lax
from jax.experimental import pallas as pl
from jax.experimental.pallas import tpu as pltpu
from jax.experimental.pallas import tpu_sc as plsc

PI = float(np.pi)
DOWN_RATIO = 4.0
INPUT_W = 1280.0
INPUT_H = 384.0
DEPTH_MIN, DEPTH_MAX = 0.1, 100.0

_DIM_MEAN = ((4.83899871, 1.80778956, 2.11565798),
             (0.91986743, 1.75302337, 0.86220807),
             (1.78652745, 1.76500989, 0.83395625))

_ATAN_C = (0.9998660, -0.3302995, 0.1801410, -0.0851330, 0.0208351)

_L = 16
_NW = 32
_SUB = 80


def _atan_poly(r):
    a = jnp.abs(r)
    big = a > 1.0
    t = jnp.where(big, 1.0 / a, a)
    t2 = t * t
    p = jnp.full_like(t, _ATAN_C[4])
    for c in (_ATAN_C[3], _ATAN_C[2], _ATAN_C[1], _ATAN_C[0]):
        p = p * t2 + c
    p = p * t
    p = jnp.where(big, (PI / 2) - p, p)
    return jnp.where(r < 0.0, -p, p)


@functools.cache
def _build(N: int):
    f32 = jnp.float32
    i32 = jnp.int32
    G = N // _L
    GPW = -(-G // _NW)
    ROWS = GPW * _L
    NSUB = ROWS // _SUB
    GPS = _SUB // _L
    mesh = plsc.VectorSubcoreMesh(core_axis_name="c", subcore_axis_name="s",
                                  num_cores=2, num_subcores=16)

    out_type = (jax.ShapeDtypeStruct((N, 4), f32),
                jax.ShapeDtypeStruct((N, 3), f32),
                jax.ShapeDtypeStruct((N,), f32),
                jax.ShapeDtypeStruct((N,), f32))
    inbuf = lambda: [pltpu.VMEM((_SUB, 2), f32),
                     pltpu.VMEM((_SUB, 4), f32),
                     pltpu.VMEM((_SUB, 3), f32),
                     pltpu.VMEM((_SUB, 16), f32)]
    scratch = (inbuf() + inbuf() + [
        pltpu.VMEM((_SUB, 4), f32),
        pltpu.VMEM((_SUB, 3), f32),
        pltpu.VMEM((ROWS,), f32),
        pltpu.VMEM((ROWS,), i32),
        pltpu.VMEM((ROWS,), f32),
        pltpu.VMEM((ROWS,), f32),
        pltpu.SemaphoreType.DMA,
        pltpu.SemaphoreType.DMA,
    ])

    @functools.partial(
        pl.kernel, out_type=out_type, mesh=mesh, scratch_types=scratch,
        compiler_params=pltpu.CompilerParams(needs_layout_passes=False))
    def _k(cen_h, po_h, do_h, dep_h, vo_h, cls_h,
           bo_h, dm_h, dp_h, al_h, *refs):
        bufs = (refs[0:4], refs[4:8])
        bo_v, dm_v, dep_v, cls_v, dp_v, al_v, isem, osem = refs[8:]

        w = lax.axis_index("s") * 2 + lax.axis_index("c")
        base = jnp.minimum(w * ROWS, N - ROWS)

        def start_in(sc, bset):
            ssl = pl.ds(base + sc * _SUB, _SUB)
            return [pltpu.async_copy(cen_h.at[ssl], bset[0], isem),
                    pltpu.async_copy(po_h.at[ssl], bset[1], isem),
                    pltpu.async_copy(do_h.at[ssl], bset[2], isem),
                    pltpu.async_copy(vo_h.at[ssl], bset[3], isem)]

        d1 = pltpu.async_copy(dep_h.at[pl.ds(base, ROWS)], dep_v, isem)
        d2 = pltpu.async_copy(cls_h.at[pl.ds(base, ROWS)], cls_v, isem)
        pend = {0: start_in(0, bufs[0]), 1: start_in(1, bufs[1])}
        d1.wait()
        d2.wait()

        iota = lax.iota(i32, _L)
        outs = []
        for sc in range(NSUB):
            cen_v, po_v, do_v, vo_v = bufs[sc % 2]
            for d in pend.pop(sc):
                d.wait()
            for d in outs:
                d.wait()

            def group(g, carry):
                off = sc * _SUB + g * _L
                lrows = g * _L + iota

                def gat(ref, col):
                    return plsc.load_gather(
                        ref, [lrows, jnp.full((_L,), col, i32)])

                def put(ref, col, v):
                    plsc.store_scatter(
                        ref, [lrows, jnp.full((_L,), col, i32)], v)

                cx = gat(cen_v, 0)
                cy = gat(cen_v, 1)
                x1 = (cx - gat(po_v, 0)) * DOWN_RATIO
                y1 = (cy - gat(po_v, 1)) * DOWN_RATIO
                x2 = (cx + gat(po_v, 2)) * DOWN_RATIO
                y2 = (cy + gat(po_v, 3)) * DOWN_RATIO
                put(bo_v, 0, jnp.clip(x1, 0.0, INPUT_W))
                put(bo_v, 1, jnp.clip(y1, 0.0, INPUT_H))
                put(bo_v, 2, x2)
                put(bo_v, 3, y2)

                cls16 = cls_v[pl.ds(off, _L)]
                is0 = cls16 == 0
                is1 = cls16 == 1
                for j in range(3):
                    mj = jnp.where(is0, _DIM_MEAN[0][j],
                                   jnp.where(is1, _DIM_MEAN[1][j],
                                             _DIM_MEAN[2][j]))
                    put(dm_v, j, jnp.exp(gat(do_v, j)) * mj)

                dp_v[pl.ds(off, _L)] = jnp.clip(
                    jnp.exp(-dep_v[pl.ds(off, _L)]), DEPTH_MIN, DEPTH_MAX)

                m = gat(vo_v, 1) - gat(vo_v, 0)
                best = jnp.zeros((_L,), i32)
                for k in (1, 2, 3):
                    dk = gat(vo_v, 2 * k + 1) - gat(vo_v, 2 * k)
                    gt = dk > m
                    m = jnp.where(gt, dk, m)
                    best = jnp.where(gt, k, best)
                col0 = 8 + 2 * best
                s0 = plsc.load_gather(vo_v, [lrows, col0])
                s1 = plsc.load_gather(vo_v, [lrows, col0 + 1])
                alpha = _atan_poly(s0 / s1)
                alpha = alpha + jnp.where(best == 3, -(PI / 2),
                                          best.astype(f32) * (PI / 2))
                alpha = jnp.where(alpha > PI, alpha - 2 * PI, alpha)
                alpha = jnp.where(alpha < -PI, alpha + 2 * PI, alpha)
                al_v[pl.ds(off, _L)] = alpha
                return carry

            plsc.parallel_loop(0, GPS, 1, unroll=GPS)(
                lambda g: (group(g, 0), None)[1])

            ssl = pl.ds(base + sc * _SUB, _SUB)
            outs = [pltpu.async_copy(bo_v, bo_h.at[ssl], osem),
                    pltpu.async_copy(dm_v, dm_h.at[ssl], osem)]
            if sc + 2 < NSUB:
                pend[sc + 2] = start_in(sc + 2, bufs[sc % 2])

        for d in outs:
            d.wait()
        pltpu.sync_copy(dp_v, dp_h.at[pl.ds(base, ROWS)])
        pltpu.sync_copy(al_v, al_h.at[pl.ds(base, ROWS)])

    return _k


def kernel(centers, pred_offset, dims_offset, depths_offset, vector_ori,
           cls_id):
    N = centers.shape[0]
    k = _build(N)
    return k(centers.astype(jnp.float32),
             pred_offset.astype(jnp.float32),
             dims_offset.astype(jnp.float32),
             depths_offset.astype(jnp.float32),
             vector_ori.astype(jnp.float32),
             cls_id.astype(jnp.int32))


# speedup vs baseline: 1.0027x; 1.0027x over previous
import functools

import jax
import jax.numpy as jnp
import numpy as np
from jax import lax
from jax.experimental import pallas as pl
from jax.experimental.pallas import tpu as pltpu
from jax.experimental.pallas import tpu_sc as plsc

PI = float(np.pi)
DOWN_RATIO = 4.0
INPUT_W = 1280.0
INPUT_H = 384.0
DEPTH_MIN, DEPTH_MAX = 0.1, 100.0

_DIM_MEAN = ((4.83899871, 1.80778956, 2.11565798),
             (0.91986743, 1.75302337, 0.86220807),
             (1.78652745, 1.76500989, 0.83395625))

_ATAN_C = (0.9998660, -0.3302995, 0.1801410, -0.0851330, 0.0208351)

_L = 16
_NW = 32
_SUB = 80


def _atan_poly(r):
    a = jnp.abs(r)
    big = a > 1.0
    t = jnp.where(big, 1.0 / a, a)
    t2 = t * t
    p = jnp.full_like(t, _ATAN_C[4])
    for c in (_ATAN_C[3], _ATAN_C[2], _ATAN_C[1], _ATAN_C[0]):
        p = p * t2 + c
    p = p * t
    p = jnp.where(big, (PI / 2) - p, p)
    return jnp.where(r < 0.0, -p, p)


@functools.cache
def _build(N: int):
    f32 = jnp.float32
    i32 = jnp.int32
    G = N // _L
    GPW = -(-G // _NW)
    ROWS = GPW * _L
    NSUB = ROWS // _SUB
    GPS = _SUB // _L
    mesh = plsc.VectorSubcoreMesh(core_axis_name="c", subcore_axis_name="s",
                                  num_cores=2, num_subcores=16)

    out_type = (jax.ShapeDtypeStruct((N, 4), f32),
                jax.ShapeDtypeStruct((N, 3), f32),
                jax.ShapeDtypeStruct((N,), f32),
                jax.ShapeDtypeStruct((N,), f32))
    inbuf = lambda: [pltpu.VMEM((_SUB, 2), f32),
                     pltpu.VMEM((_SUB, 4), f32),
                     pltpu.VMEM((_SUB, 3), f32),
                     pltpu.VMEM((_SUB, 16), f32)]
    scratch = (inbuf() + inbuf() + [
        pltpu.VMEM((2 * _SUB, 4), f32),
        pltpu.VMEM((2 * _SUB, 3), f32),
        pltpu.VMEM((ROWS,), f32),
        pltpu.VMEM((ROWS,), i32),
        pltpu.VMEM((ROWS,), f32),
        pltpu.VMEM((ROWS,), f32),
        pltpu.SemaphoreType.DMA,
        pltpu.SemaphoreType.DMA,
    ])

    @functools.partial(
        pl.kernel, out_type=out_type, mesh=mesh, scratch_types=scratch,
        compiler_params=pltpu.CompilerParams(needs_layout_passes=False))
    def _k(cen_h, po_h, do_h, dep_h, vo_h, cls_h,
           bo_h, dm_h, dp_h, al_h, *refs):
        bufs = (refs[0:4], refs[4:8])
        bo_v, dm_v, dep_v, cls_v, dp_v, al_v, isem, osem = refs[8:]

        w = lax.axis_index("s") * 2 + lax.axis_index("c")
        base = jnp.minimum(w * ROWS, N - ROWS)

        def start_in(sc, bset):
            ssl = pl.ds(base + sc * _SUB, _SUB)
            return [pltpu.async_copy(cen_h.at[ssl], bset[0], isem),
                    pltpu.async_copy(po_h.at[ssl], bset[1], isem),
                    pltpu.async_copy(do_h.at[ssl], bset[2], isem),
                    pltpu.async_copy(vo_h.at[ssl], bset[3], isem)]

        d1 = pltpu.async_copy(dep_h.at[pl.ds(base, ROWS)], dep_v, isem)
        d2 = pltpu.async_copy(cls_h.at[pl.ds(base, ROWS)], cls_v, isem)
        pend = {0: start_in(0, bufs[0]), 1: start_in(1, bufs[1])}
        d1.wait()
        d2.wait()

        iota = lax.iota(i32, _L)
        outs = []
        for sc in range(NSUB):
            cen_v, po_v, do_v, vo_v = bufs[sc % 2]
            for d in pend.pop(sc):
                d.wait()
            if sc % 2 == 0:
                for d in outs:
                    d.wait()

            def group(g, carry):
                off = sc * _SUB + g * _L
                lrows = g * _L + iota
                orows = (sc % 2) * _SUB + g * _L + iota

                def gat(ref, col):
                    return plsc.load_gather(
                        ref, [lrows, jnp.full((_L,), col, i32)])

                def put(ref, col, v):
                    plsc.store_scatter(
                        ref, [orows, jnp.full((_L,), col, i32)], v)

                cx = gat(cen_v, 0)
                cy = gat(cen_v, 1)
                x1 = (cx - gat(po_v, 0)) * DOWN_RATIO
                y1 = (cy - gat(po_v, 1)) * DOWN_RATIO
                x2 = (cx + gat(po_v, 2)) * DOWN_RATIO
                y2 = (cy + gat(po_v, 3)) * DOWN_RATIO
                put(bo_v, 0, jnp.clip(x1, 0.0, INPUT_W))
                put(bo_v, 1, jnp.clip(y1, 0.0, INPUT_H))
                put(bo_v, 2, x2)
                put(bo_v, 3, y2)

                cls16 = cls_v[pl.ds(off, _L)]
                is0 = cls16 == 0
                is1 = cls16 == 1
                for j in range(3):
                    mj = jnp.where(is0, _DIM_MEAN[0][j],
                                   jnp.where(is1, _DIM_MEAN[1][j],
                                             _DIM_MEAN[2][j]))
                    put(dm_v, j, jnp.exp(gat(do_v, j)) * mj)

                dp_v[pl.ds(off, _L)] = jnp.clip(
                    jnp.exp(-dep_v[pl.ds(off, _L)]), DEPTH_MIN, DEPTH_MAX)

                m = gat(vo_v, 1) - gat(vo_v, 0)
                best = jnp.zeros((_L,), i32)
                for k in (1, 2, 3):
                    dk = gat(vo_v, 2 * k + 1) - gat(vo_v, 2 * k)
                    gt = dk > m
                    m = jnp.where(gt, dk, m)
                    best = jnp.where(gt, k, best)
                col0 = 8 + 2 * best
                s0 = plsc.load_gather(vo_v, [lrows, col0])
                s1 = plsc.load_gather(vo_v, [lrows, col0 + 1])
                alpha = _atan_poly(s0 / s1)
                alpha = alpha + jnp.where(best == 3, -(PI / 2),
                                          best.astype(f32) * (PI / 2))
                alpha = jnp.where(alpha > PI, alpha - 2 * PI, alpha)
                alpha = jnp.where(alpha < -PI, alpha + 2 * PI, alpha)
                al_v[pl.ds(off, _L)] = alpha
                return carry

            lax.fori_loop(0, GPS, group, 0)

            if sc % 2 == 1:
                ssl = pl.ds(base + (sc - 1) * _SUB, 2 * _SUB)
                outs = [pltpu.async_copy(bo_v, bo_h.at[ssl], osem),
                        pltpu.async_copy(dm_v, dm_h.at[ssl], osem)]
            if sc + 2 < NSUB:
                pend[sc + 2] = start_in(sc + 2, bufs[sc % 2])

        for d in outs:
            d.wait()
        pltpu.sync_copy(dp_v, dp_h.at[pl.ds(base, ROWS)])
        pltpu.sync_copy(al_v, al_h.at[pl.ds(base, ROWS)])

    return _k


def kernel(centers, pred_offset, dims_offset, depths_offset, vector_ori,
           cls_id):
    N = centers.shape[0]
    k = _build(N)
    return k(centers.astype(jnp.float32),
             pred_offset.astype(jnp.float32),
             dims_offset.astype(jnp.float32),
             depths_offset.astype(jnp.float32),
             vector_ori.astype(jnp.float32),
             cls_id.astype(jnp.int32))


# speedup vs baseline: 1.0096x; 1.0069x over previous
import functools

import jax
import jax.numpy as jnp
import numpy as np
from jax import lax
from jax.experimental import pallas as pl
from jax.experimental.pallas import tpu as pltpu
from jax.experimental.pallas import tpu_sc as plsc

PI = float(np.pi)
DOWN_RATIO = 4.0
INPUT_W = 1280.0
INPUT_H = 384.0
DEPTH_MIN, DEPTH_MAX = 0.1, 100.0

_DIM_MEAN = ((4.83899871, 1.80778956, 2.11565798),
             (0.91986743, 1.75302337, 0.86220807),
             (1.78652745, 1.76500989, 0.83395625))

_ATAN_C = (0.9998660, -0.3302995, 0.1801410, -0.0851330, 0.0208351)

_L = 16
_NW = 32
_SUB = 80


def _atan_poly(r):
    a = jnp.abs(r)
    big = a > 1.0
    t = jnp.where(big, 1.0 / a, a)
    t2 = t * t
    p = jnp.full_like(t, _ATAN_C[4])
    for c in (_ATAN_C[3], _ATAN_C[2], _ATAN_C[1], _ATAN_C[0]):
        p = p * t2 + c
    p = p * t
    p = jnp.where(big, (PI / 2) - p, p)
    return jnp.where(r < 0.0, -p, p)


@functools.cache
def _build(N: int):
    f32 = jnp.float32
    i32 = jnp.int32
    G = N // _L
    GPW = -(-G // _NW)
    ROWS = GPW * _L
    NSUB = ROWS // _SUB
    GPS = _SUB // _L
    mesh = plsc.VectorSubcoreMesh(core_axis_name="c", subcore_axis_name="s",
                                  num_cores=2, num_subcores=16)

    out_type = (jax.ShapeDtypeStruct((N, 4), f32),
                jax.ShapeDtypeStruct((N, 3), f32),
                jax.ShapeDtypeStruct((N,), f32),
                jax.ShapeDtypeStruct((N,), f32))
    inbuf = lambda: [pltpu.VMEM((_SUB, 2), f32),
                     pltpu.VMEM((_SUB, 4), f32),
                     pltpu.VMEM((_SUB, 3), f32),
                     pltpu.VMEM((_SUB, 16), f32)]
    scratch = (inbuf() + inbuf() + [
        pltpu.VMEM((_SUB, 4), f32),
        pltpu.VMEM((_SUB, 3), f32),
        pltpu.VMEM((ROWS,), f32),
        pltpu.VMEM((ROWS,), i32),
        pltpu.VMEM((ROWS,), f32),
        pltpu.VMEM((ROWS,), f32),
        pltpu.SemaphoreType.DMA,
        pltpu.SemaphoreType.DMA,
    ])

    @functools.partial(
        pl.kernel, out_type=out_type, mesh=mesh, scratch_types=scratch,
        compiler_params=pltpu.CompilerParams(needs_layout_passes=False))
    def _k(cen_h, po_h, do_h, dep_h, vo_h, cls_h,
           bo_h, dm_h, dp_h, al_h, *refs):
        bufs = (refs[0:4], refs[4:8])
        bo_v, dm_v, dep_v, cls_v, dp_v, al_v, isem, osem = refs[8:]

        w = lax.axis_index("s") * 2 + lax.axis_index("c")
        base = jnp.minimum(w * ROWS, N - ROWS)

        def start_in(sc, bset):
            ssl = pl.ds(base + sc * _SUB, _SUB)
            return [pltpu.async_copy(cen_h.at[ssl], bset[0], isem),
                    pltpu.async_copy(po_h.at[ssl], bset[1], isem),
                    pltpu.async_copy(do_h.at[ssl], bset[2], isem),
                    pltpu.async_copy(vo_h.at[ssl], bset[3], isem)]

        d1 = pltpu.async_copy(dep_h.at[pl.ds(base, ROWS)], dep_v, isem)
        d2 = pltpu.async_copy(cls_h.at[pl.ds(base, ROWS)], cls_v, isem)
        pend = {0: start_in(0, bufs[0]), 1: start_in(1, bufs[1])}
        d1.wait()
        d2.wait()

        iota = lax.iota(i32, _L)
        outs = []
        for sc in range(NSUB):
            cen_v, po_v, do_v, vo_v = bufs[sc % 2]
            for d in pend.pop(sc):
                d.wait()
            for d in outs:
                d.wait()

            def group(g, carry):
                off = sc * _SUB + g * _L
                lrows = g * _L + iota

                def gat(ref, col):
                    return plsc.load_gather(
                        ref, [lrows, jnp.full((_L,), col, i32)])

                def put(ref, col, v):
                    plsc.store_scatter(
                        ref, [lrows, jnp.full((_L,), col, i32)], v)

                cx = gat(cen_v, 0)
                cy = gat(cen_v, 1)
                x1 = (cx - gat(po_v, 0)) * DOWN_RATIO
                y1 = (cy - gat(po_v, 1)) * DOWN_RATIO
                x2 = (cx + gat(po_v, 2)) * DOWN_RATIO
                y2 = (cy + gat(po_v, 3)) * DOWN_RATIO
                put(bo_v, 0, jnp.clip(x1, 0.0, INPUT_W))
                put(bo_v, 1, jnp.clip(y1, 0.0, INPUT_H))
                put(bo_v, 2, x2)
                put(bo_v, 3, y2)

                cls16 = cls_v[pl.ds(off, _L)]
                is0 = cls16 == 0
                is1 = cls16 == 1
                for j in range(3):
                    mj = jnp.where(is0, _DIM_MEAN[0][j],
                                   jnp.where(is1, _DIM_MEAN[1][j],
                                             _DIM_MEAN[2][j]))
                    put(dm_v, j, jnp.exp(gat(do_v, j)) * mj)

                dp_v[pl.ds(off, _L)] = jnp.clip(
                    jnp.exp(-dep_v[pl.ds(off, _L)]), DEPTH_MIN, DEPTH_MAX)

                m = gat(vo_v, 1) - gat(vo_v, 0)
                best = jnp.zeros((_L,), i32)
                for k in (1, 2, 3):
                    dk = gat(vo_v, 2 * k + 1) - gat(vo_v, 2 * k)
                    gt = dk > m
                    m = jnp.where(gt, dk, m)
                    best = jnp.where(gt, k, best)
                col0 = 8 + 2 * best
                s0 = plsc.load_gather(vo_v, [lrows, col0])
                s1 = plsc.load_gather(vo_v, [lrows, col0 + 1])
                alpha = _atan_poly(s0 / s1)
                alpha = alpha + jnp.where(best == 3, -(PI / 2),
                                          best.astype(f32) * (PI / 2))
                alpha = jnp.where(alpha > PI, alpha - 2 * PI, alpha)
                alpha = jnp.where(alpha < -PI, alpha + 2 * PI, alpha)
                al_v[pl.ds(off, _L)] = alpha
                return carry

            lax.fori_loop(0, GPS, group, 0)

            ssl = pl.ds(base + sc * _SUB, _SUB)
            outs = [pltpu.async_copy(bo_v, bo_h.at[ssl], osem),
                    pltpu.async_copy(dm_v, dm_h.at[ssl], osem)]
            if sc + 2 < NSUB:
                pend[sc + 2] = start_in(sc + 2, bufs[sc % 2])

        for d in outs:
            d.wait()
        pltpu.sync_copy(dp_v, dp_h.at[pl.ds(base, ROWS)])
        pltpu.sync_copy(al_v, al_h.at[pl.ds(base, ROWS)])

    return _k


def kernel(centers, pred_offset, dims_offset, depths_offset, vector_ori,
           cls_id):
    N = centers.shape[0]
    k = _build(N)
    return k(centers.astype(jnp.float32),
             pred_offset.astype(jnp.float32),
             dims_offset.astype(jnp.float32),
             depths_offset.astype(jnp.float32),
             vector_ori.astype(jnp.float32),
             cls_id.astype(jnp.int32))
